# baseline (speedup 1.0000x reference)
import functools

import jax
import jax.numpy as jnp
from jax import lax
from jax.experimental import pallas as pl
from jax.experimental.pallas import tpu as pltpu
from jax.experimental.pallas import tpu_sc as plsc

_CHUNK = 200


def kernel(emb_h, h, e, conv_w, conv_b, w2, b2, edge_index):
    E, H = e.shape
    info = plsc.get_sparse_core_info()
    nc, ns = info.num_cores, info.num_subcores
    nw = nc * ns
    rows_pw = E // nw
    n = rows_pw // _CHUNK

    mesh = plsc.VectorSubcoreMesh(core_axis_name="c", subcore_axis_name="s")

    @functools.partial(
        pl.kernel,
        out_type=jax.ShapeDtypeStruct((E, H), e.dtype),
        mesh=mesh,
        scratch_types=(
            [pltpu.VMEM((_CHUNK, H), e.dtype)] * 2
            + [pltpu.SemaphoreType.DMA] * 4
        ),
    )
    def sc_copy(e_hbm, out_hbm, b0, b1, si0, si1, so0, so1):
        bufs = (b0, b1)
        sin = (si0, si1)
        sout = (so0, so1)
        wid = lax.axis_index("s") * nc + lax.axis_index("c")
        base = wid * rows_pw
        C = _CHUNK

        def rd(m):
            return pltpu.async_copy(
                e_hbm.at[pl.ds(base + m * C, C)], bufs[m % 2], sin[m % 2])

        def wr(m):
            return pltpu.async_copy(
                bufs[m % 2], out_hbm.at[pl.ds(base + m * C, C)], sout[m % 2])

        def wait_rd(m, moff):
            pltpu.make_async_copy(
                e_hbm.at[pl.ds(base + moff * C, C)],
                bufs[m % 2], sin[m % 2]).wait()

        def wait_wr(m, moff):
            pltpu.make_async_copy(
                bufs[m % 2],
                out_hbm.at[pl.ds(base + moff * C, C)], sout[m % 2]).wait()

        rd(0)
        rd(1)
        wait_rd(0, 0)
        wr(0)

        @pl.loop(1, n - 1, step=2)
        def _body(j):
            for k in range(2):
                moff = j + k
                m = 1 - k
                wait_wr(m + 1, moff - 1)
                rd_m = moff + 1
                pltpu.async_copy(
                    e_hbm.at[pl.ds(base + rd_m * C, C)],
                    bufs[(m + 1) % 2], sin[(m + 1) % 2])
                wait_rd(m, moff)
                pltpu.async_copy(
                    bufs[m % 2],
                    out_hbm.at[pl.ds(base + moff * C, C)], sout[m % 2])

        wait_wr(0, n - 2)
        wait_rd(1, n - 1)
        wr(n - 1)
        wait_wr(1, n - 1)

    return sc_copy(e)


# speedup vs baseline: 1.3065x; 1.3065x over previous
import jax
import jax.numpy as jnp
from jax.experimental import pallas as pl
from jax.experimental.pallas import tpu as pltpu

_BLOCK_ROWS = 20000


def _copy_body(e_ref, o_ref):
    o_ref[...] = e_ref[...]


def kernel(emb_h, h, e, conv_w, conv_b, w2, b2, edge_index):
    E, H = e.shape
    block_rows = _BLOCK_ROWS if E % _BLOCK_ROWS == 0 else E
    grid = (E // block_rows,)
    out = pl.pallas_call(
        _copy_body,
        grid=grid,
        in_specs=[pl.BlockSpec((block_rows, H), lambda i: (i, 0))],
        out_specs=pl.BlockSpec((block_rows, H), lambda i: (i, 0)),
        out_shape=jax.ShapeDtypeStruct((E, H), e.dtype),
    )(e)
    return out
